# Initial kernel scaffold; baseline (speedup 1.0000x reference)
import functools

import jax
import jax.numpy as jnp
from jax import lax
from jax.experimental import pallas as pl
from jax.experimental.pallas import tpu as pltpu
from jax.experimental.pallas import tpu_sc as plsc

_NC = 2
_NS = 16
_LANES = 16
_CH = 80
_G = 128


def _sc_scatter(t2, src, dst, n_nodes):
    E = src.shape[0]
    hh = t2.shape[1]
    per_tile = E // _NS
    chunks = per_tile // _CH
    rows_per_tile = n_nodes // _NS
    zr = 125
    zchunks = rows_per_tile // zr
    mesh = plsc.VectorSubcoreMesh(core_axis_name="c", subcore_axis_name="s")

    @functools.partial(
        pl.kernel,
        mesh=mesh,
        out_type=jax.ShapeDtypeStruct((_NC, n_nodes, hh), jnp.float32),
        scratch_types=[
            pltpu.VMEM((_CH,), jnp.int32),
            pltpu.VMEM((_CH,), jnp.int32),
            pltpu.VMEM((_CH, hh), jnp.float32),
            pltpu.VMEM((zr, hh), jnp.float32),
            pltpu.VMEM_SHARED((n_nodes, hh), jnp.float32),
            pltpu.SemaphoreType.DMA,
        ],
    )
    def k(t2_hbm, src_hbm, dst_hbm, out_hbm, sidx, didx, rows, zbuf, acc, sem):
        c = lax.axis_index("c")
        s = lax.axis_index("s")
        row0 = s * rows_per_tile

        def zero_row(i, carry):
            for j in range(hh // _LANES):
                zbuf[i, pl.ds(j * _LANES, _LANES)] = jnp.zeros(
                    (_LANES,), jnp.float32)
            return carry
        lax.fori_loop(0, zr, zero_row, 0)
        for q in range(zchunks):
            pltpu.sync_copy(zbuf, acc.at[pl.ds(row0 + q * zr, zr)])
        plsc.subcore_barrier()

        def body(kk, carry):
            eb = s * per_tile + kk * _CH
            pltpu.sync_copy(src_hbm.at[pl.ds(eb, _CH)], sidx)
            pltpu.sync_copy(dst_hbm.at[pl.ds(eb, _CH)], didx)
            for j in range(_CH // _LANES):
                v = sidx[pl.ds(j * _LANES, _LANES)]
                sidx[pl.ds(j * _LANES, _LANES)] = v * 2 + c
            pltpu.async_copy(t2_hbm.at[sidx], rows, sem).wait()
            pltpu.sync_copy(rows, acc.at[didx], add=True)
            return carry
        lax.fori_loop(0, chunks, body, 0)

        plsc.subcore_barrier()
        pltpu.sync_copy(acc.at[pl.ds(row0, rows_per_tile)],
                        out_hbm.at[c, pl.ds(row0, rows_per_tile)])

    return k(t2, src, dst)


def _tc_init(X, W0, b0, Wg0):
    n, d = X.shape
    hdim = W0.shape[1]
    R = 2000
    grid = (n // R,)

    def body(x_ref, w0_ref, b0_ref, wg_ref, h_ref, t_ref):
        hb = jnp.maximum(
            jnp.dot(x_ref[...], w0_ref[...],
                    preferred_element_type=jnp.float32) + b0_ref[...], 0.0)
        h_ref[...] = hb
        t_ref[...] = jnp.dot(hb, wg_ref[...],
                             preferred_element_type=jnp.float32)

    return pl.pallas_call(
        body,
        grid=grid,
        in_specs=[
            pl.BlockSpec((R, d), lambda i: (i, 0)),
            pl.BlockSpec((d, hdim), lambda i: (0, 0)),
            pl.BlockSpec((1, hdim), lambda i: (0, 0)),
            pl.BlockSpec((hdim, hdim), lambda i: (0, 0)),
        ],
        out_specs=[
            pl.BlockSpec((R, hdim), lambda i: (i, 0)),
            pl.BlockSpec((R, hdim), lambda i: (i, 0)),
        ],
        out_shape=[
            jax.ShapeDtypeStruct((n, hdim), jnp.float32),
            jax.ShapeDtypeStruct((n, hdim), jnp.float32),
        ],
    )(X, W0, b0.reshape(1, hdim), Wg0)


def _tc_gru(m2, h, Wt0, Wt1, W_hhT, b_ih, b_hh, Wgn):
    n, hdim = h.shape
    hh = m2.shape[2]
    R = 2000
    grid = (n // R,)

    def body(m_ref, h_ref, wt0, wt1, whh, bi, bh, wg, hn_ref, t_ref):
        gi = (jnp.dot(m_ref[0], wt0[...], preferred_element_type=jnp.float32)
              + jnp.dot(m_ref[1], wt1[...], preferred_element_type=jnp.float32)
              + bi[...])
        gh = jnp.dot(h_ref[...], whh[...],
                     preferred_element_type=jnp.float32) + bh[...]
        ir, iz, inn = gi[:, :hdim], gi[:, hdim:2 * hdim], gi[:, 2 * hdim:]
        hr, hz, hn = gh[:, :hdim], gh[:, hdim:2 * hdim], gh[:, 2 * hdim:]
        r = jax.nn.sigmoid(ir + hr)
        z = jax.nn.sigmoid(iz + hz)
        nn_ = jnp.tanh(inn + r * hn)
        hnew = (1.0 - z) * nn_ + z * h_ref[...]
        hn_ref[...] = hnew
        t_ref[...] = jnp.dot(hnew, wg[...],
                             preferred_element_type=jnp.float32)

    return pl.pallas_call(
        body,
        grid=grid,
        in_specs=[
            pl.BlockSpec((_NC, R, hh), lambda i: (0, i, 0)),
            pl.BlockSpec((R, hdim), lambda i: (i, 0)),
            pl.BlockSpec((hh, 3 * hdim), lambda i: (0, 0)),
            pl.BlockSpec((hh, 3 * hdim), lambda i: (0, 0)),
            pl.BlockSpec((hdim, 3 * hdim), lambda i: (0, 0)),
            pl.BlockSpec((1, 3 * hdim), lambda i: (0, 0)),
            pl.BlockSpec((1, 3 * hdim), lambda i: (0, 0)),
            pl.BlockSpec((hdim, hdim), lambda i: (0, 0)),
        ],
        out_specs=[
            pl.BlockSpec((R, hdim), lambda i: (i, 0)),
            pl.BlockSpec((R, hdim), lambda i: (i, 0)),
        ],
        out_shape=[
            jax.ShapeDtypeStruct((n, hdim), jnp.float32),
            jax.ShapeDtypeStruct((n, hdim), jnp.float32),
        ],
    )(m2, h, Wt0, Wt1, W_hhT, b_ih.reshape(1, -1), b_hh.reshape(1, -1), Wgn)


def _tc_final(m2, h, Wt0, Wt1, W_hhT, b_ih, b_hh,
              W1, b1, W2, b2, W3, b3, W4, b4, batch):
    n, hdim = h.shape
    hh = m2.shape[2]
    R = 2000
    nb = n // R
    grid = (nb,)
    batch3 = batch.reshape(nb, 1, R)

    def body(m_ref, h_ref, wt0, wt1, whh, bi, bh,
             w1, b1r, w2, b2r, w3, b3r, w4, b4r, seg_ref,
             logits_ref, sums_ref, cnts_ref):
        i = pl.program_id(0)
        gi = (jnp.dot(m_ref[0], wt0[...], preferred_element_type=jnp.float32)
              + jnp.dot(m_ref[1], wt1[...], preferred_element_type=jnp.float32)
              + bi[...])
        gh = jnp.dot(h_ref[...], whh[...],
                     preferred_element_type=jnp.float32) + bh[...]
        ir, iz, inn = gi[:, :hdim], gi[:, hdim:2 * hdim], gi[:, 2 * hdim:]
        hr, hz, hn = gh[:, :hdim], gh[:, hdim:2 * hdim], gh[:, 2 * hdim:]
        r = jax.nn.sigmoid(ir + hr)
        z = jax.nn.sigmoid(iz + hz)
        nn_ = jnp.tanh(inn + r * hn)
        hnew = (1.0 - z) * nn_ + z * h_ref[...]
        x = jnp.maximum(hnew, 0.0)
        x = jnp.maximum(jnp.dot(x, w1[...],
                                preferred_element_type=jnp.float32)
                        + b1r[...], 0.0)
        x = jnp.maximum(jnp.dot(x, w2[...],
                                preferred_element_type=jnp.float32)
                        + b2r[...], 0.0)
        x = jnp.maximum(jnp.dot(x, w3[...],
                                preferred_element_type=jnp.float32)
                        + b3r[...], 0.0)
        seg = seg_ref[0]
        onehot = (lax.broadcasted_iota(jnp.int32, (_G, R), 0)
                  == seg).astype(jnp.float32)
        psum = jnp.dot(onehot, x, preferred_element_type=jnp.float32)
        pcnt = jnp.sum(onehot, axis=1, keepdims=True)

        @pl.when(i == 0)
        def _():
            sums_ref[...] = jnp.zeros_like(sums_ref)
            cnts_ref[...] = jnp.zeros_like(cnts_ref)

        sums_ref[...] += psum
        cnts_ref[...] += pcnt

        @pl.when(i == nb - 1)
        def _():
            pooled = sums_ref[...] / jnp.maximum(cnts_ref[...], 1.0)
            logits_ref[...] = jnp.dot(
                pooled, w4[...], preferred_element_type=jnp.float32) + b4r[...]

    return pl.pallas_call(
        body,
        grid=grid,
        in_specs=[
            pl.BlockSpec((_NC, R, hh), lambda i: (0, i, 0)),
            pl.BlockSpec((R, hdim), lambda i: (i, 0)),
            pl.BlockSpec((hh, 3 * hdim), lambda i: (0, 0)),
            pl.BlockSpec((hh, 3 * hdim), lambda i: (0, 0)),
            pl.BlockSpec((hdim, 3 * hdim), lambda i: (0, 0)),
            pl.BlockSpec((1, 3 * hdim), lambda i: (0, 0)),
            pl.BlockSpec((1, 3 * hdim), lambda i: (0, 0)),
            pl.BlockSpec((hdim, 256), lambda i: (0, 0)),
            pl.BlockSpec((1, 256), lambda i: (0, 0)),
            pl.BlockSpec((256, 128), lambda i: (0, 0)),
            pl.BlockSpec((1, 128), lambda i: (0, 0)),
            pl.BlockSpec((128, 256), lambda i: (0, 0)),
            pl.BlockSpec((1, 256), lambda i: (0, 0)),
            pl.BlockSpec((256, 1), lambda i: (0, 0)),
            pl.BlockSpec((1, 1), lambda i: (0, 0)),
            pl.BlockSpec((1, 1, R), lambda i: (i, 0, 0)),
        ],
        out_specs=pl.BlockSpec((_G, 1), lambda i: (0, 0)),
        out_shape=jax.ShapeDtypeStruct((_G, 1), jnp.float32),
        scratch_shapes=[
            pltpu.VMEM((_G, 256), jnp.float32),
            pltpu.VMEM((_G, 1), jnp.float32),
        ],
    )(m2, h, Wt0, Wt1, W_hhT, b_ih.reshape(1, -1), b_hh.reshape(1, -1),
      W1, b1.reshape(1, -1), W2, b2.reshape(1, -1), W3, b3.reshape(1, -1),
      W4, b4.reshape(1, -1), batch3)


def kernel(X, edge_index, batch, W0, b0, Wg, W_ih, W_hh, b_ih, b_hh,
           W1, b1, W2, b2, W3, b3, W4, b4):
    n, d = X.shape
    hdim = W0.shape[1]
    hh = hdim // 2
    L = Wg.shape[0]
    src = edge_index[0]
    dst = edge_index[1]
    W_ihT = W_ih.T
    Wt0 = W_ihT[:hh]
    Wt1 = W_ihT[hh:]
    W_hhT = W_hh.T

    h, t = _tc_init(X, W0, b0, Wg[0])
    for i in range(L):
        m2 = _sc_scatter(t.reshape(2 * n, hh), src, dst, n)
        if i < L - 1:
            h, t = _tc_gru(m2, h, Wt0, Wt1, W_hhT, b_ih, b_hh, Wg[i + 1])
        else:
            logits = _tc_final(m2, h, Wt0, Wt1, W_hhT, b_ih, b_hh,
                               W1, b1, W2, b2, W3, b3, W4, b4, batch)
    return logits


# speedup vs baseline: 3.2799x; 3.2799x over previous
import functools

import jax
import jax.numpy as jnp
from jax import lax
from jax.experimental import pallas as pl
from jax.experimental.pallas import tpu as pltpu
from jax.experimental.pallas import tpu_sc as plsc

_NC = 2
_NS = 16
_LANES = 16
_CH = 80
_G = 128


def _sc_scatter(t2, src, dst, n_nodes):
    E = src.shape[0]
    hh = t2.shape[1]
    per_tile = E // _NS
    chunks = per_tile // _CH
    rfull = 632
    rlast = n_nodes - (_NS - 1) * rfull
    zr = 200
    mesh = plsc.VectorSubcoreMesh(core_axis_name="c", subcore_axis_name="s")

    @functools.partial(
        pl.kernel,
        mesh=mesh,
        out_type=jax.ShapeDtypeStruct((_NC, n_nodes, hh), jnp.float32),
        scratch_types=[
            pltpu.VMEM((_CH,), jnp.int32),
            pltpu.VMEM((_CH,), jnp.int32),
            pltpu.VMEM((_CH, hh), jnp.float32),
            pltpu.VMEM((zr, hh), jnp.float32),
            pltpu.VMEM_SHARED((n_nodes, hh), jnp.float32),
            pltpu.SemaphoreType.DMA,
        ],
    )
    def k(t2_hbm, src_hbm, dst_hbm, out_hbm, sidx, didx, rows, zbuf, acc, sem):
        c = lax.axis_index("c")
        s = lax.axis_index("s")
        row0 = s * rfull

        def zero_row(i, carry):
            for j in range(hh // _LANES):
                zbuf[i, pl.ds(j * _LANES, _LANES)] = jnp.zeros(
                    (_LANES,), jnp.float32)
            return carry
        lax.fori_loop(0, zr, zero_row, 0)

        @pl.when(s < _NS - 1)
        def _():
            for q in range(rfull // zr):
                pltpu.sync_copy(zbuf, acc.at[pl.ds(row0 + q * zr, zr)])
            rem = rfull % zr
            if rem:
                pltpu.sync_copy(zbuf.at[pl.ds(0, rem)],
                                acc.at[pl.ds(row0 + rfull - rem, rem)])

        @pl.when(s == _NS - 1)
        def _():
            for q in range(rlast // zr):
                pltpu.sync_copy(zbuf, acc.at[pl.ds(row0 + q * zr, zr)])
            rem = rlast % zr
            if rem:
                pltpu.sync_copy(zbuf.at[pl.ds(0, rem)],
                                acc.at[pl.ds(row0 + rlast - rem, rem)])

        plsc.subcore_barrier()

        def body(kk, carry):
            eb = s * per_tile + kk * _CH
            pltpu.sync_copy(src_hbm.at[pl.ds(eb, _CH)], sidx)
            pltpu.sync_copy(dst_hbm.at[pl.ds(eb, _CH)], didx)
            for j in range(_CH // _LANES):
                v = sidx[pl.ds(j * _LANES, _LANES)]
                sidx[pl.ds(j * _LANES, _LANES)] = v * 2 + c
            pltpu.async_copy(t2_hbm.at[sidx], rows, sem).wait()
            pltpu.sync_copy(rows, acc.at[didx], add=True)
            return carry
        lax.fori_loop(0, chunks, body, 0)

        plsc.subcore_barrier()

        @pl.when(s < _NS - 1)
        def _():
            pltpu.sync_copy(acc.at[pl.ds(row0, rfull)],
                            out_hbm.at[c, pl.ds(row0, rfull)])

        @pl.when(s == _NS - 1)
        def _():
            pltpu.sync_copy(acc.at[pl.ds(row0, rlast)],
                            out_hbm.at[c, pl.ds(row0, rlast)])

    return k(t2, src, dst)


def _tc_init(X, W0, b0, Wg0):
    n, d = X.shape
    hdim = W0.shape[1]
    R = 2000
    grid = (n // R,)

    def body(x_ref, w0_ref, b0_ref, wg_ref, h_ref, t_ref):
        hb = jnp.maximum(
            jnp.dot(x_ref[...], w0_ref[...],
                    preferred_element_type=jnp.float32) + b0_ref[...], 0.0)
        h_ref[...] = hb
        t_ref[...] = jnp.dot(hb, wg_ref[...],
                             preferred_element_type=jnp.float32)

    return pl.pallas_call(
        body,
        grid=grid,
        in_specs=[
            pl.BlockSpec((R, d), lambda i: (i, 0)),
            pl.BlockSpec((d, hdim), lambda i: (0, 0)),
            pl.BlockSpec((1, hdim), lambda i: (0, 0)),
            pl.BlockSpec((hdim, hdim), lambda i: (0, 0)),
        ],
        out_specs=[
            pl.BlockSpec((R, hdim), lambda i: (i, 0)),
            pl.BlockSpec((R, hdim), lambda i: (i, 0)),
        ],
        out_shape=[
            jax.ShapeDtypeStruct((n, hdim), jnp.float32),
            jax.ShapeDtypeStruct((n, hdim), jnp.float32),
        ],
    )(X, W0, b0.reshape(1, hdim), Wg0)


def _tc_gru(m2, h, Wt0, Wt1, W_hhT, b_ih, b_hh, Wgn):
    n, hdim = h.shape
    hh = m2.shape[2]
    R = 2000
    grid = (n // R,)

    def body(m_ref, h_ref, wt0, wt1, whh, bi, bh, wg, hn_ref, t_ref):
        gi = (jnp.dot(m_ref[0], wt0[...], preferred_element_type=jnp.float32)
              + jnp.dot(m_ref[1], wt1[...], preferred_element_type=jnp.float32)
              + bi[...])
        gh = jnp.dot(h_ref[...], whh[...],
                     preferred_element_type=jnp.float32) + bh[...]
        ir, iz, inn = gi[:, :hdim], gi[:, hdim:2 * hdim], gi[:, 2 * hdim:]
        hr, hz, hn = gh[:, :hdim], gh[:, hdim:2 * hdim], gh[:, 2 * hdim:]
        r = jax.nn.sigmoid(ir + hr)
        z = jax.nn.sigmoid(iz + hz)
        nn_ = jnp.tanh(inn + r * hn)
        hnew = (1.0 - z) * nn_ + z * h_ref[...]
        hn_ref[...] = hnew
        t_ref[...] = jnp.dot(hnew, wg[...],
                             preferred_element_type=jnp.float32)

    return pl.pallas_call(
        body,
        grid=grid,
        in_specs=[
            pl.BlockSpec((_NC, R, hh), lambda i: (0, i, 0)),
            pl.BlockSpec((R, hdim), lambda i: (i, 0)),
            pl.BlockSpec((hh, 3 * hdim), lambda i: (0, 0)),
            pl.BlockSpec((hh, 3 * hdim), lambda i: (0, 0)),
            pl.BlockSpec((hdim, 3 * hdim), lambda i: (0, 0)),
            pl.BlockSpec((1, 3 * hdim), lambda i: (0, 0)),
            pl.BlockSpec((1, 3 * hdim), lambda i: (0, 0)),
            pl.BlockSpec((hdim, hdim), lambda i: (0, 0)),
        ],
        out_specs=[
            pl.BlockSpec((R, hdim), lambda i: (i, 0)),
            pl.BlockSpec((R, hdim), lambda i: (i, 0)),
        ],
        out_shape=[
            jax.ShapeDtypeStruct((n, hdim), jnp.float32),
            jax.ShapeDtypeStruct((n, hdim), jnp.float32),
        ],
    )(m2, h, Wt0, Wt1, W_hhT, b_ih.reshape(1, -1), b_hh.reshape(1, -1), Wgn)


def _tc_final(m2, h, Wt0, Wt1, W_hhT, b_ih, b_hh,
              W1, b1, W2, b2, W3, b3, W4, b4, batch):
    n, hdim = h.shape
    hh = m2.shape[2]
    R = 2000
    nb = n // R
    grid = (nb,)
    batch3 = batch.reshape(nb, 1, R)

    def body(m_ref, h_ref, wt0, wt1, whh, bi, bh,
             w1, b1r, w2, b2r, w3, b3r, w4, b4r, seg_ref,
             logits_ref, sums_ref, cnts_ref):
        i = pl.program_id(0)
        gi = (jnp.dot(m_ref[0], wt0[...], preferred_element_type=jnp.float32)
              + jnp.dot(m_ref[1], wt1[...], preferred_element_type=jnp.float32)
              + bi[...])
        gh = jnp.dot(h_ref[...], whh[...],
                     preferred_element_type=jnp.float32) + bh[...]
        ir, iz, inn = gi[:, :hdim], gi[:, hdim:2 * hdim], gi[:, 2 * hdim:]
        hr, hz, hn = gh[:, :hdim], gh[:, hdim:2 * hdim], gh[:, 2 * hdim:]
        r = jax.nn.sigmoid(ir + hr)
        z = jax.nn.sigmoid(iz + hz)
        nn_ = jnp.tanh(inn + r * hn)
        hnew = (1.0 - z) * nn_ + z * h_ref[...]
        x = jnp.maximum(hnew, 0.0)
        x = jnp.maximum(jnp.dot(x, w1[...],
                                preferred_element_type=jnp.float32)
                        + b1r[...], 0.0)
        x = jnp.maximum(jnp.dot(x, w2[...],
                                preferred_element_type=jnp.float32)
                        + b2r[...], 0.0)
        x = jnp.maximum(jnp.dot(x, w3[...],
                                preferred_element_type=jnp.float32)
                        + b3r[...], 0.0)
        seg = seg_ref[0]
        onehot = (lax.broadcasted_iota(jnp.int32, (_G, R), 0)
                  == seg).astype(jnp.float32)
        psum = jnp.dot(onehot, x, preferred_element_type=jnp.float32)
        pcnt = jnp.sum(onehot, axis=1, keepdims=True)

        @pl.when(i == 0)
        def _():
            sums_ref[...] = jnp.zeros_like(sums_ref)
            cnts_ref[...] = jnp.zeros_like(cnts_ref)

        sums_ref[...] += psum
        cnts_ref[...] += pcnt

        @pl.when(i == nb - 1)
        def _():
            pooled = sums_ref[...] / jnp.maximum(cnts_ref[...], 1.0)
            logits_ref[...] = jnp.dot(
                pooled, w4[...], preferred_element_type=jnp.float32) + b4r[...]

    return pl.pallas_call(
        body,
        grid=grid,
        in_specs=[
            pl.BlockSpec((_NC, R, hh), lambda i: (0, i, 0)),
            pl.BlockSpec((R, hdim), lambda i: (i, 0)),
            pl.BlockSpec((hh, 3 * hdim), lambda i: (0, 0)),
            pl.BlockSpec((hh, 3 * hdim), lambda i: (0, 0)),
            pl.BlockSpec((hdim, 3 * hdim), lambda i: (0, 0)),
            pl.BlockSpec((1, 3 * hdim), lambda i: (0, 0)),
            pl.BlockSpec((1, 3 * hdim), lambda i: (0, 0)),
            pl.BlockSpec((hdim, 256), lambda i: (0, 0)),
            pl.BlockSpec((1, 256), lambda i: (0, 0)),
            pl.BlockSpec((256, 128), lambda i: (0, 0)),
            pl.BlockSpec((1, 128), lambda i: (0, 0)),
            pl.BlockSpec((128, 256), lambda i: (0, 0)),
            pl.BlockSpec((1, 256), lambda i: (0, 0)),
            pl.BlockSpec((256, 1), lambda i: (0, 0)),
            pl.BlockSpec((1, 1), lambda i: (0, 0)),
            pl.BlockSpec((1, 1, R), lambda i: (i, 0, 0)),
        ],
        out_specs=pl.BlockSpec((_G, 1), lambda i: (0, 0)),
        out_shape=jax.ShapeDtypeStruct((_G, 1), jnp.float32),
        scratch_shapes=[
            pltpu.VMEM((_G, 256), jnp.float32),
            pltpu.VMEM((_G, 1), jnp.float32),
        ],
    )(m2, h, Wt0, Wt1, W_hhT, b_ih.reshape(1, -1), b_hh.reshape(1, -1),
      W1, b1.reshape(1, -1), W2, b2.reshape(1, -1), W3, b3.reshape(1, -1),
      W4, b4.reshape(1, -1), batch3)


def kernel(X, edge_index, batch, W0, b0, Wg, W_ih, W_hh, b_ih, b_hh,
           W1, b1, W2, b2, W3, b3, W4, b4):
    n, d = X.shape
    hdim = W0.shape[1]
    hh = hdim // 2
    L = Wg.shape[0]
    src = edge_index[0]
    dst = edge_index[1]
    W_ihT = W_ih.T
    Wt0 = W_ihT[:hh]
    Wt1 = W_ihT[hh:]
    W_hhT = W_hh.T

    h, t = _tc_init(X, W0, b0, Wg[0])
    for i in range(L):
        m2 = _sc_scatter(t.reshape(2 * n, hh), src, dst, n)
        if i < L - 1:
            h, t = _tc_gru(m2, h, Wt0, Wt1, W_hhT, b_ih, b_hh, Wg[i + 1])
        else:
            logits = _tc_final(m2, h, Wt0, Wt1, W_hhT, b_ih, b_hh,
                               W1, b1, W2, b2, W3, b3, W4, b4, batch)
    return logits
